# Initial kernel scaffold; baseline (speedup 1.0000x reference)
import jax
import jax.numpy as jnp
from jax.experimental import pallas as pl


def kernel(hidden_states, sim_matrix, gates, q_proj, k_proj, v_proj, o_proj):
    raise NotImplementedError("write your pallas kernel here")


# speedup vs baseline: 2.6291x; 2.6291x over previous
import jax
import jax.numpy as jnp
import numpy as np
from jax.experimental import pallas as pl

B, T, C = 2, 2048, 768
E, MIN_E, HD = 16, 2, 64
BLK = 256


def _gate_qkv_body(x_ref, sim_ref, gates_ref, w_all_ref,
                   q_ref, k_ref, v_ref, wgt_ref):
    x = x_ref[...]
    xnorm = jnp.sqrt(jnp.sum(x * x, axis=1, keepdims=True))
    hn = x / jnp.maximum(xnorm, 1e-12)
    sim = sim_ref[...]
    snorm = jnp.sqrt(jnp.sum(sim * sim, axis=0, keepdims=True))
    sn = sim / jnp.maximum(snorm, 1e-12)
    logits = jnp.dot(hn, sn, preferred_element_type=jnp.float32)
    logits = logits - jax.nn.sigmoid(gates_ref[...])
    gated = jnp.maximum(logits, 0.0)
    mask = (gated > 0.0).astype(jnp.float32)
    inactive = jnp.sum(mask, axis=1, keepdims=True) == 0.0
    iota = jax.lax.broadcasted_iota(jnp.int32, logits.shape, 1)
    max1 = jnp.max(logits, axis=1, keepdims=True)
    idx1 = jnp.min(jnp.where(logits == max1, iota, E), axis=1, keepdims=True)
    l2 = jnp.where(iota == idx1, -jnp.inf, logits)
    max2 = jnp.max(l2, axis=1, keepdims=True)
    idx2 = jnp.min(jnp.where(l2 == max2, iota, E), axis=1, keepdims=True)
    fb = jnp.logical_or(iota == idx1, iota == idx2)
    mask = jnp.where(jnp.logical_and(inactive, fb), 1.0, mask)
    gm = jnp.where(mask > 0.0, gated, jnp.float32(-1e9))
    gm_max = jnp.max(gm, axis=1, keepdims=True)
    p = jnp.exp(gm - gm_max)
    probs = p / jnp.sum(p, axis=1, keepdims=True)
    wgt_ref[...] = probs * mask
    P = jnp.dot(x, w_all_ref[...], preferred_element_type=jnp.float32)
    accq = jnp.zeros((BLK, HD), jnp.float32)
    acck = jnp.zeros((BLK, HD), jnp.float32)
    accv = jnp.zeros((BLK, HD), jnp.float32)
    for e in range(E):
        m = mask[:, e:e + 1]
        accq = accq + m * P[:, e * HD:(e + 1) * HD]
        acck = acck + m * P[:, E * HD + e * HD:E * HD + (e + 1) * HD]
        accv = accv + m * P[:, 2 * E * HD + e * HD:2 * E * HD + (e + 1) * HD]
    q_ref[...] = accq
    k_ref[...] = acck
    v_ref[...] = accv


def _attn_out_body(q_ref, k_ref, v_ref, wgt_ref, ost_ref, o_ref):
    qi = pl.program_id(1)
    q = q_ref[0]
    k = k_ref[0]
    v = v_ref[0]
    scale = jnp.float32(1.0 / np.sqrt(HD))
    s = jax.lax.dot_general(q, k, (((1,), (1,)), ((), ())),
                            preferred_element_type=jnp.float32) * scale
    row = qi * BLK + jax.lax.broadcasted_iota(jnp.int32, s.shape, 0)
    col = jax.lax.broadcasted_iota(jnp.int32, s.shape, 1)
    s = jnp.where(col <= row, s, jnp.float32(-1e9))
    m = jnp.max(s, axis=1, keepdims=True)
    p = jnp.exp(s - m)
    attn = p / jnp.sum(p, axis=1, keepdims=True)
    o = jnp.dot(attn, v, preferred_element_type=jnp.float32)
    w = wgt_ref[0]
    a = jnp.concatenate([w[:, e:e + 1] * o for e in range(E)], axis=1)
    o_ref[0] = jnp.dot(a, ost_ref[...], preferred_element_type=jnp.float32)


def kernel(hidden_states, sim_matrix, gates, q_proj, k_proj, v_proj, o_proj):
    flat = hidden_states.reshape(B * T, C)
    w_all = jnp.concatenate(
        [q_proj.transpose(1, 0, 2).reshape(C, E * HD),
         k_proj.transpose(1, 0, 2).reshape(C, E * HD),
         v_proj.transpose(1, 0, 2).reshape(C, E * HD)], axis=1)
    gates2 = gates.reshape(1, E)
    nblk = (B * T) // BLK

    q, k, v, wgt = pl.pallas_call(
        _gate_qkv_body,
        grid=(nblk,),
        in_specs=[
            pl.BlockSpec((BLK, C), lambda i: (i, 0)),
            pl.BlockSpec((C, E), lambda i: (0, 0)),
            pl.BlockSpec((1, E), lambda i: (0, 0)),
            pl.BlockSpec((C, 3 * E * HD), lambda i: (0, 0)),
        ],
        out_specs=[
            pl.BlockSpec((BLK, HD), lambda i: (i, 0)),
            pl.BlockSpec((BLK, HD), lambda i: (i, 0)),
            pl.BlockSpec((BLK, HD), lambda i: (i, 0)),
            pl.BlockSpec((BLK, E), lambda i: (i, 0)),
        ],
        out_shape=[
            jax.ShapeDtypeStruct((B * T, HD), jnp.float32),
            jax.ShapeDtypeStruct((B * T, HD), jnp.float32),
            jax.ShapeDtypeStruct((B * T, HD), jnp.float32),
            jax.ShapeDtypeStruct((B * T, E), jnp.float32),
        ],
    )(flat, sim_matrix, gates2, w_all)

    q3 = q.reshape(B, T, HD)
    k3 = k.reshape(B, T, HD)
    v3 = v.reshape(B, T, HD)
    w3 = wgt.reshape(B, T, E)
    o_st = o_proj.reshape(E * HD, C)

    out = pl.pallas_call(
        _attn_out_body,
        grid=(B, T // BLK),
        in_specs=[
            pl.BlockSpec((1, BLK, HD), lambda b, i: (b, i, 0)),
            pl.BlockSpec((1, T, HD), lambda b, i: (b, 0, 0)),
            pl.BlockSpec((1, T, HD), lambda b, i: (b, 0, 0)),
            pl.BlockSpec((1, BLK, E), lambda b, i: (b, i, 0)),
            pl.BlockSpec((E * HD, C), lambda b, i: (0, 0)),
        ],
        out_specs=pl.BlockSpec((1, BLK, C), lambda b, i: (b, i, 0)),
        out_shape=jax.ShapeDtypeStruct((B, T, C), jnp.float32),
    )(q3, k3, v3, w3, o_st)

    return out


# speedup vs baseline: 2.7421x; 1.0430x over previous
import jax
import jax.numpy as jnp
import numpy as np
from jax.experimental import pallas as pl

B, T, C = 2, 2048, 768
E, MIN_E, HD = 16, 2, 64
BLK = 256


def _gate_qkv_body(x_ref, sim_ref, gates_ref, w_all_ref,
                   q_ref, k_ref, v_ref, wgt_ref):
    x = x_ref[...]
    xnorm = jnp.sqrt(jnp.sum(x * x, axis=1, keepdims=True))
    hn = x / jnp.maximum(xnorm, 1e-12)
    sim = sim_ref[...]
    snorm = jnp.sqrt(jnp.sum(sim * sim, axis=0, keepdims=True))
    sn = sim / jnp.maximum(snorm, 1e-12)
    logits = jnp.dot(hn, sn, preferred_element_type=jnp.float32)
    logits = logits - jax.nn.sigmoid(gates_ref[...])
    gated = jnp.maximum(logits, 0.0)
    mask = (gated > 0.0).astype(jnp.float32)
    inactive = jnp.sum(mask, axis=1, keepdims=True) == 0.0
    iota = jax.lax.broadcasted_iota(jnp.int32, logits.shape, 1)
    max1 = jnp.max(logits, axis=1, keepdims=True)
    idx1 = jnp.min(jnp.where(logits == max1, iota, E), axis=1, keepdims=True)
    l2 = jnp.where(iota == idx1, -jnp.inf, logits)
    max2 = jnp.max(l2, axis=1, keepdims=True)
    idx2 = jnp.min(jnp.where(l2 == max2, iota, E), axis=1, keepdims=True)
    fb = jnp.logical_or(iota == idx1, iota == idx2)
    mask = jnp.where(jnp.logical_and(inactive, fb), 1.0, mask)
    gm = jnp.where(mask > 0.0, gated, jnp.float32(-1e9))
    gm_max = jnp.max(gm, axis=1, keepdims=True)
    p = jnp.exp(gm - gm_max)
    probs = p / jnp.sum(p, axis=1, keepdims=True)
    wgt_ref[...] = probs * mask
    P = jnp.dot(x.astype(jnp.bfloat16), w_all_ref[...],
                preferred_element_type=jnp.float32)
    accq = jnp.zeros((BLK, HD), jnp.float32)
    acck = jnp.zeros((BLK, HD), jnp.float32)
    accv = jnp.zeros((BLK, HD), jnp.float32)
    for e in range(E):
        m = mask[:, e:e + 1]
        accq = accq + m * P[:, e * HD:(e + 1) * HD]
        acck = acck + m * P[:, E * HD + e * HD:E * HD + (e + 1) * HD]
        accv = accv + m * P[:, 2 * E * HD + e * HD:2 * E * HD + (e + 1) * HD]
    q_ref[...] = accq
    k_ref[...] = acck
    v_ref[...] = accv


def _attn_out_body(q_ref, k_ref, v_ref, wgt_ref, ost_ref, o_ref):
    qi = pl.program_id(1)
    q = q_ref[0].astype(jnp.bfloat16)
    k = k_ref[0].astype(jnp.bfloat16)
    v = v_ref[0].astype(jnp.bfloat16)
    scale = jnp.float32(1.0 / np.sqrt(HD))
    s = jax.lax.dot_general(q, k, (((1,), (1,)), ((), ())),
                            preferred_element_type=jnp.float32) * scale
    row = qi * BLK + jax.lax.broadcasted_iota(jnp.int32, s.shape, 0)
    col = jax.lax.broadcasted_iota(jnp.int32, s.shape, 1)
    s = jnp.where(col <= row, s, jnp.float32(-1e9))
    m = jnp.max(s, axis=1, keepdims=True)
    p = jnp.exp(s - m)
    attn = p / jnp.sum(p, axis=1, keepdims=True)
    o = jnp.dot(attn.astype(jnp.bfloat16), v,
                preferred_element_type=jnp.float32)
    w = wgt_ref[0]
    a = jnp.concatenate([w[:, e:e + 1] * o for e in range(E)], axis=1)
    o_ref[0] = jnp.dot(a.astype(jnp.bfloat16), ost_ref[...],
                       preferred_element_type=jnp.float32)


def kernel(hidden_states, sim_matrix, gates, q_proj, k_proj, v_proj, o_proj):
    flat = hidden_states.reshape(B * T, C)
    w_all = jnp.concatenate(
        [q_proj.transpose(1, 0, 2).reshape(C, E * HD),
         k_proj.transpose(1, 0, 2).reshape(C, E * HD),
         v_proj.transpose(1, 0, 2).reshape(C, E * HD)],
        axis=1).astype(jnp.bfloat16)
    gates2 = gates.reshape(1, E)
    nblk = (B * T) // BLK

    q, k, v, wgt = pl.pallas_call(
        _gate_qkv_body,
        grid=(nblk,),
        in_specs=[
            pl.BlockSpec((BLK, C), lambda i: (i, 0)),
            pl.BlockSpec((C, E), lambda i: (0, 0)),
            pl.BlockSpec((1, E), lambda i: (0, 0)),
            pl.BlockSpec((C, 3 * E * HD), lambda i: (0, 0)),
        ],
        out_specs=[
            pl.BlockSpec((BLK, HD), lambda i: (i, 0)),
            pl.BlockSpec((BLK, HD), lambda i: (i, 0)),
            pl.BlockSpec((BLK, HD), lambda i: (i, 0)),
            pl.BlockSpec((BLK, E), lambda i: (i, 0)),
        ],
        out_shape=[
            jax.ShapeDtypeStruct((B * T, HD), jnp.float32),
            jax.ShapeDtypeStruct((B * T, HD), jnp.float32),
            jax.ShapeDtypeStruct((B * T, HD), jnp.float32),
            jax.ShapeDtypeStruct((B * T, E), jnp.float32),
        ],
    )(flat, sim_matrix, gates2, w_all)

    q3 = q.reshape(B, T, HD)
    k3 = k.reshape(B, T, HD)
    v3 = v.reshape(B, T, HD)
    w3 = wgt.reshape(B, T, E)
    o_st = o_proj.reshape(E * HD, C).astype(jnp.bfloat16)

    out = pl.pallas_call(
        _attn_out_body,
        grid=(B, T // BLK),
        in_specs=[
            pl.BlockSpec((1, BLK, HD), lambda b, i: (b, i, 0)),
            pl.BlockSpec((1, T, HD), lambda b, i: (b, 0, 0)),
            pl.BlockSpec((1, T, HD), lambda b, i: (b, 0, 0)),
            pl.BlockSpec((1, BLK, E), lambda b, i: (b, i, 0)),
            pl.BlockSpec((E * HD, C), lambda b, i: (0, 0)),
        ],
        out_specs=pl.BlockSpec((1, BLK, C), lambda b, i: (b, i, 0)),
        out_shape=jax.ShapeDtypeStruct((B, T, C), jnp.float32),
    )(q3, k3, v3, w3, o_st)

    return out
